# speedup vs baseline: 1.2827x; 1.2827x over previous
import functools

import jax
import jax.numpy as jnp
from jax import lax
from jax.experimental import pallas as pl
from jax.experimental.pallas import tpu as pltpu
from jax.experimental.pallas import tpu_sc as plsc

D = 768
B_TOTAL = 8192
L = 16
NC = 2
NS = 16
NW = NC * NS
B_PER_W = B_TOTAL // NW
C = 64
NCHUNK = B_PER_W // C

_mesh = plsc.VectorSubcoreMesh(core_axis_name="c", subcore_axis_name="s")


@functools.partial(
    pl.kernel,
    mesh=_mesh,
    out_type=jax.ShapeDtypeStruct((B_TOTAL, D), jnp.float32),
    scratch_types=[
        pltpu.VMEM((C,), jnp.int32),
        pltpu.VMEM((C,), jnp.int32),
        pltpu.VMEM((C, D), jnp.float32),
        pltpu.VMEM((C, D), jnp.float32),
        pltpu.SemaphoreType.DMA,
        pltpu.SemaphoreType.DMA,
    ],
)
def _embed_lookup(ids_hbm, pids_hbm, wtab_hbm, ptab_hbm, out_hbm,
                  widx, pidx, wrows, prows, wsem, psem):
    wid = lax.axis_index("s") * NC + lax.axis_index("c")
    base = wid * B_PER_W
    for c in range(NCHUNK):
        off = base + c * C
        pltpu.sync_copy(ids_hbm.at[pl.ds(off, C)], widx)
        pltpu.sync_copy(pids_hbm.at[pl.ds(off, C)], pidx)
        wcp = pltpu.async_copy(wtab_hbm.at[widx], wrows, wsem)
        pcp = pltpu.async_copy(ptab_hbm.at[pidx], prows, psem)
        wcp.wait()
        pcp.wait()

        def add_row(r, carry):
            for j in range(D // L):
                s = wrows[r, pl.ds(j * L, L)] + prows[r, pl.ds(j * L, L)]
                wrows[r, pl.ds(j * L, L)] = s
            return carry

        lax.fori_loop(0, C, add_row, 0)
        pltpu.sync_copy(wrows, out_hbm.at[pl.ds(off, C)])


def kernel(input_ids, position_ids, attention_mask, word_embeddings,
           position_embeddings):
    b, s = input_ids.shape
    ids = input_ids.reshape(-1).astype(jnp.int32)
    pids = position_ids.reshape(-1).astype(jnp.int32)
    out = _embed_lookup(ids, pids, word_embeddings, position_embeddings)
    return out.reshape(b, s, D)


# speedup vs baseline: 1.6266x; 1.2681x over previous
import functools

import jax
import jax.numpy as jnp
from jax import lax
from jax.experimental import pallas as pl
from jax.experimental.pallas import tpu as pltpu
from jax.experimental.pallas import tpu_sc as plsc

D = 768
B_TOTAL = 8192
L = 16
NC = 2
NS = 16
NW = NC * NS
B_PER_W = B_TOTAL // NW
C = 32
NCHUNK = B_PER_W // C

_mesh = plsc.VectorSubcoreMesh(core_axis_name="c", subcore_axis_name="s")


@functools.partial(
    pl.kernel,
    mesh=_mesh,
    out_type=jax.ShapeDtypeStruct((B_TOTAL, D), jnp.float32),
    scratch_types=[
        pltpu.VMEM((B_PER_W,), jnp.int32),
        pltpu.VMEM((B_PER_W,), jnp.int32),
        pltpu.VMEM((C, D), jnp.float32),
        pltpu.VMEM((C, D), jnp.float32),
        pltpu.VMEM((C, D), jnp.float32),
        pltpu.VMEM((C, D), jnp.float32),
        pltpu.SemaphoreType.DMA,
        pltpu.SemaphoreType.DMA,
        pltpu.SemaphoreType.DMA,
        pltpu.SemaphoreType.DMA,
        pltpu.SemaphoreType.DMA,
        pltpu.SemaphoreType.DMA,
    ],
)
def _embed_lookup(ids_hbm, pids_hbm, wtab_hbm, ptab_hbm, out_hbm,
                  widx, pidx, wr0, wr1, pr0, pr1,
                  ws0, ws1, ps0, ps1, os0, os1):
    wid = lax.axis_index("s") * NC + lax.axis_index("c")
    base = wid * B_PER_W
    pltpu.sync_copy(ids_hbm.at[pl.ds(base, B_PER_W)], widx)
    pltpu.sync_copy(pids_hbm.at[pl.ds(base, B_PER_W)], pidx)

    wbuf = (wr0, wr1)
    pbuf = (pr0, pr1)
    wsem = (ws0, ws1)
    psem = (ps0, ps1)
    osem = (os0, os1)

    def issue_gathers(g, b):
        wd = pltpu.async_copy(
            wtab_hbm.at[widx.at[pl.ds(g * C, C)]], wbuf[b], wsem[b])
        pd = pltpu.async_copy(
            ptab_hbm.at[pidx.at[pl.ds(g * C, C)]], pbuf[b], psem[b])
        return wd, pd

    wd = [None] * NCHUNK
    pd = [None] * NCHUNK
    od = [None] * NCHUNK
    wd[0], pd[0] = issue_gathers(0, 0)

    for g in range(NCHUNK):
        b = g & 1
        if g + 1 < NCHUNK:
            if g >= 1:
                od[g - 1].wait()
            wd[g + 1], pd[g + 1] = issue_gathers(g + 1, 1 - b)
        wd[g].wait()
        pd[g].wait()

        def add_row(r, carry, _wb=wbuf[b], _pb=pbuf[b]):
            for j in range(D // L):
                s = _wb[r, pl.ds(j * L, L)] + _pb[r, pl.ds(j * L, L)]
                _wb[r, pl.ds(j * L, L)] = s
            return carry

        lax.fori_loop(0, C, add_row, 0)
        od[g] = pltpu.async_copy(
            wbuf[b], out_hbm.at[pl.ds(base + g * C, C)], osem[b])

    od[NCHUNK - 2].wait()
    od[NCHUNK - 1].wait()


def kernel(input_ids, position_ids, attention_mask, word_embeddings,
           position_embeddings):
    b, s = input_ids.shape
    ids = input_ids.reshape(-1).astype(jnp.int32)
    pids = position_ids.reshape(-1).astype(jnp.int32)
    out = _embed_lookup(ids, pids, word_embeddings, position_embeddings)
    return out.reshape(b, s, D)
